# baseline (speedup 1.0000x reference)
import functools

import numpy as np

import jax
import jax.numpy as jnp
from jax import lax
from jax.experimental import pallas as pl
from jax.experimental.pallas import tpu as pltpu
from jax.experimental.pallas import tpu_sc as plsc

NC = 2
NS = 16
NW = NC * NS
B = 128
R = 2000


def _cdiv(a, b):
    return (a + b - 1) // b


def _sc_mesh():
    return plsc.VectorSubcoreMesh(core_axis_name="c", subcore_axis_name="s",
                                  num_cores=NC, num_subcores=NS)


def _make_hist(nblk, npad, pw):
    zr = npad // NW
    grp = 8
    ngrp = pw // grp

    @functools.partial(
        pl.kernel,
        out_type=jax.ShapeDtypeStruct((NC, npad, 8), jnp.float32),
        mesh=_sc_mesh(),
        scratch_types=[
            pltpu.VMEM((pw, B), jnp.int32),
            pltpu.VMEM((B, 8), jnp.float32),
            pltpu.VMEM_SHARED((npad, 8), jnp.float32),
            pltpu.SemaphoreType.DMA,
        ],
    )
    def hist(idx_hbm, ones_hbm, zrow_hbm, out_hbm, idx_all, ones_v, acc, sem):
        cid = lax.axis_index("c")
        sid = lax.axis_index("s")
        wid = sid * NC + cid
        row0 = sid * zr
        pltpu.sync_copy(zrow_hbm, acc.at[pl.ds(row0, zr)])
        pltpu.sync_copy(ones_hbm, ones_v)
        pltpu.sync_copy(idx_hbm.at[1, pl.ds(wid * pw, pw)], idx_all)
        plsc.subcore_barrier()

        def fire(g):
            for b in range(grp):
                pltpu.async_copy(ones_v, acc.at[idx_all.at[g * grp + b]],
                                 sem, add=True)

        fire(0)

        def body(g, carry):
            @pl.when(g + 1 < ngrp)
            def _():
                fire(g + 1)

            for b in range(grp):
                pltpu.make_async_copy(ones_v, acc.at[idx_all.at[0]],
                                      sem).wait()
            return carry

        lax.fori_loop(0, ngrp, body, 0)
        plsc.subcore_barrier()
        pltpu.sync_copy(acc.at[pl.ds(row0, zr)],
                        out_hbm.at[cid, pl.ds(row0, zr)])

    return hist


def _make_agg(n, d, nblk, npad, pw):
    zr = npad // NW
    budget = (2097151 - npad * d) // NS - 2 * B * d - 256
    cb = 1
    for c in range(1, pw + 1):
        if pw % c == 0 and c * 2 * B <= budget:
            cb = c
    nch = pw // cb

    @functools.partial(
        pl.kernel,
        out_type=jax.ShapeDtypeStruct((NC, npad, d), jnp.float32),
        mesh=_sc_mesh(),
        scratch_types=[
            pltpu.VMEM((2, cb, B), jnp.int32),
            pltpu.VMEM((2, B, d), jnp.float32),
            pltpu.VMEM_SHARED((npad, d), jnp.float32),
            pltpu.SemaphoreType.DMA,
            pltpu.SemaphoreType.DMA,
        ],
    )
    def agg(u_hbm, idx_hbm, zrow_hbm, out_hbm, idx_all, rows2, acc,
            gsem, ssem):
        cid = lax.axis_index("c")
        sid = lax.axis_index("s")
        wid = sid * NC + cid
        row0 = sid * zr
        pltpu.sync_copy(zrow_hbm, acc.at[pl.ds(row0, zr)])
        plsc.subcore_barrier()

        def wait_gather(j):
            pltpu.make_async_copy(u_hbm.at[idx_all.at[0, 0]],
                                  rows2.at[j], gsem).wait()

        def wait_scatter(j):
            pltpu.make_async_copy(rows2.at[j], acc.at[idx_all.at[1, 0]],
                                  ssem).wait()

        def outer(c, carry):
            pltpu.sync_copy(
                idx_hbm.at[:, pl.ds(wid * pw + c * cb, cb), :], idx_all)
            pltpu.async_copy(u_hbm.at[idx_all.at[0, 0]], rows2.at[0], gsem)

            def body(i, inner):
                j = lax.rem(i, 2)
                nj = 1 - j

                @pl.when(i >= 1)
                def _():
                    wait_scatter(nj)

                @pl.when(i + 1 < cb)
                def _():
                    pltpu.async_copy(u_hbm.at[idx_all.at[0, i + 1]],
                                     rows2.at[nj], gsem)

                wait_gather(j)
                pltpu.async_copy(rows2.at[j], acc.at[idx_all.at[1, i]],
                                 ssem, add=True)
                return inner

            lax.fori_loop(0, cb, body, 0)
            wait_scatter(lax.rem(cb - 1, 2))
            return carry

        lax.fori_loop(0, nch, outer, 0)
        plsc.subcore_barrier()
        pltpu.sync_copy(acc.at[pl.ds(row0, zr)],
                        out_hbm.at[cid, pl.ds(row0, zr)])

    return agg


def _s1_body(x_ref, w_ref, hist_ref, u_ref, dinv_ref):
    deg = (hist_ref[0] + hist_ref[1]).astype(jnp.float32)
    dinv = lax.rsqrt(deg[:, 0:1] + 1.0)
    xw = jnp.dot(x_ref[...], w_ref[...], preferred_element_type=jnp.float32)
    u_ref[...] = xw * dinv
    dinv_ref[...] = dinv


def _s2_body(p_ref, u1_ref, dinv_ref, b_ref, w_ref, u_ref):
    dinv = dinv_ref[...]
    s = (p_ref[0] + p_ref[1] + u1_ref[...]) * dinv + b_ref[...]
    h = jnp.maximum(s, 0.0)
    u_ref[...] = jnp.dot(h, w_ref[...], preferred_element_type=jnp.float32) * dinv


def _s3_body(q_ref, u2_ref, dinv_ref, bm_ref, bl_ref, zm_ref, zl_ref):
    dz = bm_ref.shape[1]
    s = (q_ref[0] + q_ref[1] + u2_ref[...]) * dinv_ref[...]
    zm_ref[...] = s[:, :dz] + bm_ref[...]
    zl_ref[...] = s[:, dz:] + bl_ref[...]


def kernel(x, edge_index, W1, b1, W_mu, b_mu, W_ls, b_ls):
    n, d_in = x.shape
    d_h = W1.shape[1]
    d_z = W_mu.shape[1]
    e = edge_index.shape[1]

    npad = NW * 8 * _cdiv(n + 1, NW * 8)
    pw = 8 * _cdiv(e, NW * B * 8)
    nblk = pw * NW
    epad = nblk * B
    grid = n // R

    pad = epad - e
    pad_cols = np.stack([np.arange(pad, dtype=np.int32) % n,
                         np.full((pad,), n, dtype=np.int32)])
    eip = jnp.concatenate([edge_index, jnp.asarray(pad_cols)],
                          axis=1).reshape(2, nblk, B)

    ones8 = jnp.ones((B, 8), jnp.float32)
    zrow8 = jnp.zeros((npad // NW, 8), jnp.float32)
    zrowd = jnp.zeros((npad // NW, d_h), jnp.float32)

    hist = _make_hist(nblk, npad, pw)(eip, ones8, zrow8)

    u1, dinv = pl.pallas_call(
        _s1_body,
        grid=(grid,),
        in_specs=[
            pl.BlockSpec((R, d_in), lambda i: (i, 0)),
            pl.BlockSpec((d_in, d_h), lambda i: (0, 0)),
            pl.BlockSpec((NC, R, 8), lambda i: (0, i, 0)),
        ],
        out_specs=[
            pl.BlockSpec((R, d_h), lambda i: (i, 0)),
            pl.BlockSpec((R, 1), lambda i: (i, 0)),
        ],
        out_shape=[
            jax.ShapeDtypeStruct((n, d_h), jnp.float32),
            jax.ShapeDtypeStruct((n, 1), jnp.float32),
        ],
    )(x, W1, hist)

    p1 = _make_agg(n, d_h, nblk, npad, pw)(u1, eip, zrowd)

    w_cat = jnp.concatenate([W_mu, W_ls], axis=1)
    u2 = pl.pallas_call(
        _s2_body,
        grid=(grid,),
        in_specs=[
            pl.BlockSpec((NC, R, d_h), lambda i: (0, i, 0)),
            pl.BlockSpec((R, d_h), lambda i: (i, 0)),
            pl.BlockSpec((R, 1), lambda i: (i, 0)),
            pl.BlockSpec((1, d_h), lambda i: (0, 0)),
            pl.BlockSpec((d_h, 2 * d_z), lambda i: (0, 0)),
        ],
        out_specs=pl.BlockSpec((R, 2 * d_z), lambda i: (i, 0)),
        out_shape=jax.ShapeDtypeStruct((n, 2 * d_z), jnp.float32),
    )(p1, u1, dinv, b1.reshape(1, -1), w_cat)

    p2 = _make_agg(n, 2 * d_z, nblk, npad, pw)(u2, eip, zrowd)

    z_mu, z_ls = pl.pallas_call(
        _s3_body,
        grid=(grid,),
        in_specs=[
            pl.BlockSpec((NC, R, 2 * d_z), lambda i: (0, i, 0)),
            pl.BlockSpec((R, 2 * d_z), lambda i: (i, 0)),
            pl.BlockSpec((R, 1), lambda i: (i, 0)),
            pl.BlockSpec((1, d_z), lambda i: (0, 0)),
            pl.BlockSpec((1, d_z), lambda i: (0, 0)),
        ],
        out_specs=[
            pl.BlockSpec((R, d_z), lambda i: (i, 0)),
            pl.BlockSpec((R, d_z), lambda i: (i, 0)),
        ],
        out_shape=[
            jax.ShapeDtypeStruct((n, d_z), jnp.float32),
            jax.ShapeDtypeStruct((n, d_z), jnp.float32),
        ],
    )(p2, u2, dinv, b_mu.reshape(1, -1), b_ls.reshape(1, -1))

    return (z_mu, z_ls)


# speedup vs baseline: 1.0230x; 1.0230x over previous
import functools

import numpy as np

import jax
import jax.numpy as jnp
from jax import lax
from jax.experimental import pallas as pl
from jax.experimental.pallas import tpu as pltpu
from jax.experimental.pallas import tpu_sc as plsc

NC = 2
NS = 16
NW = NC * NS
B = 125
R = 2000


def _cdiv(a, b):
    return (a + b - 1) // b


def _sc_mesh():
    return plsc.VectorSubcoreMesh(core_axis_name="c", subcore_axis_name="s",
                                  num_cores=NC, num_subcores=NS)


def _make_hist(nblk, npad, pw):
    zr = npad // NW
    grp = 8
    ngrp = pw // grp

    @functools.partial(
        pl.kernel,
        out_type=jax.ShapeDtypeStruct((NC, npad, 8), jnp.float32),
        mesh=_sc_mesh(),
        scratch_types=[
            pltpu.VMEM((pw, B), jnp.int32),
            pltpu.VMEM((B, 8), jnp.float32),
            pltpu.VMEM_SHARED((npad, 8), jnp.float32),
            pltpu.SemaphoreType.DMA,
        ],
    )
    def hist(idx_hbm, ones_hbm, zrow_hbm, out_hbm, idx_all, ones_v, acc, sem):
        cid = lax.axis_index("c")
        sid = lax.axis_index("s")
        wid = sid * NC + cid
        row0 = sid * zr
        pltpu.sync_copy(zrow_hbm, acc.at[pl.ds(row0, zr)])
        pltpu.sync_copy(ones_hbm, ones_v)
        pltpu.sync_copy(idx_hbm.at[1, pl.ds(wid * pw, pw)], idx_all)
        plsc.subcore_barrier()

        def fire(g):
            for b in range(grp):
                pltpu.async_copy(ones_v, acc.at[idx_all.at[g * grp + b]],
                                 sem, add=True)

        fire(0)

        def body(g, carry):
            @pl.when(g + 1 < ngrp)
            def _():
                fire(g + 1)

            for b in range(grp):
                pltpu.make_async_copy(ones_v, acc.at[idx_all.at[0]],
                                      sem).wait()
            return carry

        lax.fori_loop(0, ngrp, body, 0)
        plsc.subcore_barrier()
        pltpu.sync_copy(acc.at[pl.ds(row0, zr)],
                        out_hbm.at[cid, pl.ds(row0, zr)])

    return hist


def _make_agg(n, d, nblk, npad, pw):
    zr = npad // NW
    budget = (2097151 - npad * d) // NS - 2 * B * d - 256
    cb = 1
    for c in range(1, pw + 1):
        if pw % c == 0 and c * 2 * B <= budget:
            cb = c
    nch = pw // cb

    @functools.partial(
        pl.kernel,
        out_type=jax.ShapeDtypeStruct((NC, npad, d), jnp.float32),
        mesh=_sc_mesh(),
        scratch_types=[
            pltpu.VMEM((2, cb, B), jnp.int32),
            pltpu.VMEM((2, B, d), jnp.float32),
            pltpu.VMEM_SHARED((npad, d), jnp.float32),
            pltpu.SemaphoreType.DMA,
            pltpu.SemaphoreType.DMA,
        ],
    )
    def agg(u_hbm, idx_hbm, zrow_hbm, out_hbm, idx_all, rows2, acc,
            gsem, ssem):
        cid = lax.axis_index("c")
        sid = lax.axis_index("s")
        wid = sid * NC + cid
        row0 = sid * zr
        pltpu.sync_copy(zrow_hbm, acc.at[pl.ds(row0, zr)])
        plsc.subcore_barrier()

        def wait_gather(j):
            pltpu.make_async_copy(u_hbm.at[idx_all.at[0, 0]],
                                  rows2.at[j], gsem).wait()

        def wait_scatter(j):
            pltpu.make_async_copy(rows2.at[j], acc.at[idx_all.at[1, 0]],
                                  ssem).wait()

        def outer(c, carry):
            pltpu.sync_copy(
                idx_hbm.at[:, pl.ds(wid * pw + c * cb, cb), :], idx_all)
            pltpu.async_copy(u_hbm.at[idx_all.at[0, 0]], rows2.at[0], gsem)

            def body(i, inner):
                j = lax.rem(i, 2)
                nj = 1 - j

                @pl.when(i >= 1)
                def _():
                    wait_scatter(nj)

                @pl.when(i + 1 < cb)
                def _():
                    pltpu.async_copy(u_hbm.at[idx_all.at[0, i + 1]],
                                     rows2.at[nj], gsem)

                wait_gather(j)
                pltpu.async_copy(rows2.at[j], acc.at[idx_all.at[1, i]],
                                 ssem, add=True)
                return inner

            lax.fori_loop(0, cb, body, 0)
            wait_scatter(lax.rem(cb - 1, 2))
            return carry

        lax.fori_loop(0, nch, outer, 0)
        plsc.subcore_barrier()
        pltpu.sync_copy(acc.at[pl.ds(row0, zr)],
                        out_hbm.at[cid, pl.ds(row0, zr)])

    return agg


def _s1_body(x_ref, w_ref, hist_ref, u_ref, dinv_ref):
    deg = (hist_ref[0] + hist_ref[1]).astype(jnp.float32)
    dinv = lax.rsqrt(deg[:, 0:1] + 1.0)
    xw = jnp.dot(x_ref[...], w_ref[...], preferred_element_type=jnp.float32)
    u_ref[...] = xw * dinv
    dinv_ref[...] = dinv


def _s2_body(p_ref, u1_ref, dinv_ref, b_ref, w_ref, u_ref):
    dinv = dinv_ref[...]
    s = (p_ref[0] + p_ref[1] + u1_ref[...]) * dinv + b_ref[...]
    h = jnp.maximum(s, 0.0)
    u_ref[...] = jnp.dot(h, w_ref[...], preferred_element_type=jnp.float32) * dinv


def _s3_body(q_ref, u2_ref, dinv_ref, bm_ref, bl_ref, zm_ref, zl_ref):
    dz = bm_ref.shape[1]
    s = (q_ref[0] + q_ref[1] + u2_ref[...]) * dinv_ref[...]
    zm_ref[...] = s[:, :dz] + bm_ref[...]
    zl_ref[...] = s[:, dz:] + bl_ref[...]


def kernel(x, edge_index, W1, b1, W_mu, b_mu, W_ls, b_ls):
    n, d_in = x.shape
    d_h = W1.shape[1]
    d_z = W_mu.shape[1]
    e = edge_index.shape[1]

    npad = NW * 8 * _cdiv(n + 1, NW * 8)
    pw = _cdiv(e, NW * B)
    nblk = pw * NW
    epad = nblk * B
    grid = n // R

    pad = epad - e
    if pad:
        pad_cols = np.stack([np.arange(pad, dtype=np.int32) % n,
                             np.full((pad,), n, dtype=np.int32)])
        eip = jnp.concatenate([edge_index, jnp.asarray(pad_cols)], axis=1)
    else:
        eip = edge_index
    eip = eip.reshape(2, nblk, B)

    ones8 = jnp.ones((B, 8), jnp.float32)
    zrow8 = jnp.zeros((npad // NW, 8), jnp.float32)
    zrowd = jnp.zeros((npad // NW, d_h), jnp.float32)

    hist = _make_hist(nblk, npad, pw)(eip, ones8, zrow8)

    u1, dinv = pl.pallas_call(
        _s1_body,
        grid=(grid,),
        in_specs=[
            pl.BlockSpec((R, d_in), lambda i: (i, 0)),
            pl.BlockSpec((d_in, d_h), lambda i: (0, 0)),
            pl.BlockSpec((NC, R, 8), lambda i: (0, i, 0)),
        ],
        out_specs=[
            pl.BlockSpec((R, d_h), lambda i: (i, 0)),
            pl.BlockSpec((R, 1), lambda i: (i, 0)),
        ],
        out_shape=[
            jax.ShapeDtypeStruct((n, d_h), jnp.float32),
            jax.ShapeDtypeStruct((n, 1), jnp.float32),
        ],
    )(x, W1, hist)

    p1 = _make_agg(n, d_h, nblk, npad, pw)(u1, eip, zrowd)

    w_cat = jnp.concatenate([W_mu, W_ls], axis=1)
    u2 = pl.pallas_call(
        _s2_body,
        grid=(grid,),
        in_specs=[
            pl.BlockSpec((NC, R, d_h), lambda i: (0, i, 0)),
            pl.BlockSpec((R, d_h), lambda i: (i, 0)),
            pl.BlockSpec((R, 1), lambda i: (i, 0)),
            pl.BlockSpec((1, d_h), lambda i: (0, 0)),
            pl.BlockSpec((d_h, 2 * d_z), lambda i: (0, 0)),
        ],
        out_specs=pl.BlockSpec((R, 2 * d_z), lambda i: (i, 0)),
        out_shape=jax.ShapeDtypeStruct((n, 2 * d_z), jnp.float32),
    )(p1, u1, dinv, b1.reshape(1, -1), w_cat)

    p2 = _make_agg(n, 2 * d_z, nblk, npad, pw)(u2, eip, zrowd)

    z_mu, z_ls = pl.pallas_call(
        _s3_body,
        grid=(grid,),
        in_specs=[
            pl.BlockSpec((NC, R, 2 * d_z), lambda i: (0, i, 0)),
            pl.BlockSpec((R, 2 * d_z), lambda i: (i, 0)),
            pl.BlockSpec((R, 1), lambda i: (i, 0)),
            pl.BlockSpec((1, d_z), lambda i: (0, 0)),
            pl.BlockSpec((1, d_z), lambda i: (0, 0)),
        ],
        out_specs=[
            pl.BlockSpec((R, d_z), lambda i: (i, 0)),
            pl.BlockSpec((R, d_z), lambda i: (i, 0)),
        ],
        out_shape=[
            jax.ShapeDtypeStruct((n, d_z), jnp.float32),
            jax.ShapeDtypeStruct((n, d_z), jnp.float32),
        ],
    )(p2, u2, dinv, b_mu.reshape(1, -1), b_ls.reshape(1, -1))

    return (z_mu, z_ls)
